# baseline (speedup 1.0000x reference)
import functools

import jax
import jax.numpy as jnp
from jax import lax
from jax.experimental import pallas as pl
from jax.experimental.pallas import tpu as pltpu
from jax.experimental.pallas import tpu_sc as plsc

B = 16384
F = 64
N_ROWS = 1000000
NUM_CORES = 2
NUM_SUBCORES = 16
NW = NUM_CORES * NUM_SUBCORES
BPW = B // NW
NBLK = (N_ROWS + 127) // 128
NBW = (NBLK + NW - 1) // NW
LANES = 16
NVREG = B // LANES
RING = 8

_mesh = plsc.VectorSubcoreMesh(core_axis_name="c", subcore_axis_name="s")
_params = pltpu.CompilerParams(needs_layout_passes=False,
                               use_tc_tiling_on_sc=True)


@functools.partial(
    pl.kernel,
    out_type=jax.ShapeDtypeStruct((2, B, 128), jnp.float32),
    mesh=_mesh,
    scratch_types=[
        pltpu.VMEM((2 * (B + 16),), jnp.int32),
        pltpu.VMEM((2 * (B + 16),), jnp.int32),
        pltpu.SMEM((2, 257), jnp.int32),
        pltpu.SMEM((2, 256), jnp.int32),
        pltpu.VMEM((6, F, 128), jnp.float32),
        pltpu.VMEM((RING, 128), jnp.float32),
    ] + [pltpu.SemaphoreType.DMA] * 6 + [pltpu.SemaphoreType.DMA] * RING,
    compiler_params=_params,
)
def _mf_extract(user_hbm, item_hbm, utab_hbm, itab_hbm, out_hbm,
                aidx, mkeys, ends, nblist, slab, cols, *sems):
    slab_sems = sems[:6]
    ring_sems = sems[6:]
    wid = lax.axis_index("s") * NUM_CORES + lax.axis_index("c")
    t0 = wid * NBW
    nt = jnp.minimum(NBW, NBLK - t0)

    B16 = B + 16
    pltpu.sync_copy(user_hbm, aidx.at[pl.ds(0, B)])
    pltpu.sync_copy(item_hbm, aidx.at[pl.ds(B16, B)])

    lane = lax.iota(jnp.int32, 16)

    m_per_tab = []
    for tab in range(2):
        def scan_body(v, off, tab=tab):
            idx16 = aidx[pl.ds(tab * B16 + v * LANES, LANES)]
            rel = lax.shift_right_logical(idx16, 7) - t0
            msk = (rel >= 0) & (rel < nt)
            key = (lax.shift_left(rel, 21)
                   | lax.shift_left(idx16 & 127, 14)
                   | (v * LANES + lane))
            pos = plsc.cumsum(msk.astype(jnp.int32))
            plsc.store_scatter(mkeys, [tab * B16 + off + pos - 1], key,
                               mask=msk)
            return off + pos[15]
        m_per_tab.append(lax.fori_loop(0, NVREG, scan_body, jnp.int32(0)))

    for tab in range(2):
        def zero_body(k, c, tab=tab):
            ends[tab, k] = 0
            return c
        lax.fori_loop(0, 257, zero_body, 0)

        def count_body(j, c, tab=tab):
            key = mkeys[pl.ds(tab * B16 + j, 16)][0]
            t = lax.shift_right_logical(key, 21)
            ends[tab, t] = ends[tab, t] + 1
            return c
        lax.fori_loop(0, m_per_tab[tab], count_body, 0)

        def prefix_body(k, s, tab=tab):
            s2 = s + ends[tab, k]
            ends[tab, k] = s2
            return s2
        lax.fori_loop(0, nt, prefix_body, jnp.int32(0))

        def place_body(j, c, tab=tab):
            key = mkeys[pl.ds(tab * B16 + j, 16)][0]
            t = lax.shift_right_logical(key, 21)
            p = ends[tab, t] - 1
            ends[tab, t] = p
            plsc.store_scatter(aidx, [jnp.full((16,), tab * B16, jnp.int32) + p],
                               jnp.full((16,), key, jnp.int32),
                               mask=(lane == 0))
            return c
        lax.fori_loop(0, m_per_tab[tab], place_body, 0)
        ends[tab, nt] = m_per_tab[tab]

        def nb_body(t, k, tab=tab):
            nonempty = ends[tab, t + 1] > ends[tab, t]

            @pl.when(nonempty)
            def _add():
                nblist[tab, k] = t
            return k + jnp.where(nonempty, 1, 0)
        ends[tab, 256] = lax.fori_loop(0, nt, nb_body, jnp.int32(0))

    for tab in range(2):
        tab_hbm = utab_hbm if tab == 0 else itab_hbm
        nc = ends[tab, 256]

        def fire_chunk(c, r, tab=tab, tab_hbm=tab_hbm):
            t = nblist[tab, c]
            off = pl.multiple_of((t0 + t) * 128, 128)
            for rr in range(6):
                @pl.when(r == rr)
                def _f(rr=rr):
                    pltpu.make_async_copy(
                        tab_hbm.at[:, pl.ds(off, 128)], slab.at[rr],
                        slab_sems[rr]).start()

        def wait_chunk(r, tab_hbm=tab_hbm):
            for rr in range(6):
                @pl.when(r == rr)
                def _w(rr=rr):
                    pltpu.make_async_copy(
                        tab_hbm.at[:, pl.ds(0, 128)], slab.at[rr],
                        slab_sems[rr]).wait()

        for k in range(5):
            @pl.when(k < nc)
            def _prime(k=k):
                fire_chunk(jnp.int32(k), jnp.int32(k))

        def chunk_body(c, used, tab=tab, tab_hbm=tab_hbm):
            r = lax.rem(c, 6)

            @pl.when(c + 5 < nc)
            def _fire_ahead():
                fire_chunk(c + 5, lax.rem(c + 5, 6))

            wait_chunk(r)

            t = nblist[tab, c]
            lo = ends[tab, t]
            hi = ends[tab, t + 1]
            bufv = jnp.full((16,), 0, jnp.int32) + r

            def group_body(g, used2, tab=tab):
                j0 = lo + g * RING
                new_used = used2
                for s in range(RING):
                    j = j0 + s

                    @pl.when(j < hi)
                    def _elem(s=s, j=j):
                        @pl.when((used2 >> s) & 1 == 1)
                        def _reuse_wait():
                            pltpu.make_async_copy(
                                cols.at[s], out_hbm.at[tab, 0],
                                ring_sems[s]).wait()

                        key = aidx[pl.ds(tab * B16 + j, 16)][0]
                        l = lax.shift_right_logical(key, 14) & 127
                        b = key & 16383
                        lv = jnp.full((16,), 0, jnp.int32) + l
                        for q in range(4):
                            jv = q * 16 + lane
                            vals = plsc.load_gather(slab, [bufv, jv, lv])
                            cols[s, pl.ds(q * 16, 16)] = vals
                        pltpu.make_async_copy(
                            cols.at[s], out_hbm.at[tab, b],
                            ring_sems[s]).start()

                    new_used = jnp.where(j < hi, new_used | (1 << s),
                                         new_used)
                return new_used

            n_groups = lax.div(hi - lo + (RING - 1), RING)
            return lax.fori_loop(0, n_groups, group_body, used)

        used_f = lax.fori_loop(0, nc, chunk_body, jnp.int32(0))

        for s in range(RING):
            @pl.when((used_f >> s) & 1 == 1)
            def _drain(s=s):
                pltpu.make_async_copy(
                    cols.at[s], out_hbm.at[tab, 0], ring_sems[s]).wait()


@functools.partial(
    pl.kernel,
    out_type=jax.ShapeDtypeStruct((B,), jnp.float32),
    mesh=_mesh,
    scratch_types=[
        pltpu.VMEM((128, 128), jnp.float32),
        pltpu.VMEM((128, 128), jnp.float32),
        pltpu.VMEM((BPW,), jnp.float32),
    ],
    compiler_params=_params,
)
def _mf_dot(packed_hbm, out_hbm, uslab, islab, outv):
    wid = lax.axis_index("s") * NUM_CORES + lax.axis_index("c")
    base = wid * BPW
    lane = lax.iota(jnp.int32, 16)

    for h in range(BPW // 128):
        pltpu.sync_copy(packed_hbm.at[0, pl.ds(base + h * 128, 128), :], uslab)
        pltpu.sync_copy(packed_hbm.at[1, pl.ds(base + h * 128, 128), :], islab)

        def group_body(g, carry, h=h):
            r16 = g * LANES + lane
            acc = jnp.zeros((16,), jnp.float32)
            for j in range(F):
                cj = jnp.full((16,), j, jnp.int32)
                acc = acc + (plsc.load_gather(uslab, [r16, cj])
                             * plsc.load_gather(islab, [r16, cj]))
            outv[pl.ds(h * 128 + g * LANES, LANES)] = acc
            return carry

        lax.fori_loop(0, 128 // LANES, group_body, 0)

    pltpu.sync_copy(outv, out_hbm.at[pl.ds(base, BPW)])


def kernel(user, item, user_table, item_table):
    packed = _mf_extract(user.astype(jnp.int32), item.astype(jnp.int32),
                         user_table.T, item_table.T)
    return _mf_dot(packed)


# speedup vs baseline: 1.0803x; 1.0803x over previous
import functools

import jax
import jax.numpy as jnp
from jax import lax
from jax.experimental import pallas as pl
from jax.experimental.pallas import tpu as pltpu
from jax.experimental.pallas import tpu_sc as plsc

B = 16384
F = 64
N_ROWS = 1000000
NUM_CORES = 2
NUM_SUBCORES = 16
NW = NUM_CORES * NUM_SUBCORES
BPW = B // NW
NBLK = (N_ROWS + 127) // 128
NBW = (NBLK + NW - 1) // NW
LANES = 16
NVREG = B // LANES
RING = 8

_mesh = plsc.VectorSubcoreMesh(core_axis_name="c", subcore_axis_name="s")
_params = pltpu.CompilerParams(needs_layout_passes=False,
                               use_tc_tiling_on_sc=True)


@functools.partial(
    pl.kernel,
    out_type=jax.ShapeDtypeStruct((2, B, 128), jnp.float32),
    mesh=_mesh,
    scratch_types=[
        pltpu.VMEM((2 * (B + 16),), jnp.int32),
        pltpu.VMEM((B + 16,), jnp.int32),
        pltpu.SMEM((2, 257), jnp.int32),
        pltpu.VMEM((4, F, 256), jnp.float32),
        pltpu.VMEM((RING, 128), jnp.float32),
    ] + [pltpu.SemaphoreType.DMA] * 4 + [pltpu.SemaphoreType.DMA] * RING,
    compiler_params=_params,
)
def _mf_extract(user_hbm, item_hbm, utab_hbm, itab_hbm, out_hbm,
                aidx, mkeys, ends, slab, cols, *sems):
    slab_sems = sems[:4]
    ring_sems = sems[4:]
    wid = lax.axis_index("s") * NUM_CORES + lax.axis_index("c")
    t0 = wid * NBW
    nt = jnp.minimum(NBW, NBLK - t0)

    B16 = B + 16
    pltpu.sync_copy(user_hbm, aidx.at[pl.ds(0, B)])
    pltpu.sync_copy(item_hbm, aidx.at[pl.ds(B16, B)])

    lane = lax.iota(jnp.int32, 16)

    m_per_tab = []
    for tab in range(2):
        def scan_body(v, off, tab=tab):
            idx16 = aidx[pl.ds(tab * B16 + v * LANES, LANES)]
            rel = lax.shift_right_logical(idx16, 7) - t0
            msk = (rel >= 0) & (rel < nt)
            key = (lax.shift_left(rel, 21)
                   | lax.shift_left(idx16 & 127, 14)
                   | (v * LANES + lane))
            pos = plsc.cumsum(msk.astype(jnp.int32))
            plsc.store_scatter(mkeys, [off + pos - 1], key, mask=msk)
            return off + pos[15]
        m = lax.fori_loop(0, NVREG, scan_body, jnp.int32(0))
        m_per_tab.append(m)

        def zero_body(k, c, tab=tab):
            ends[tab, k] = 0
            return c
        lax.fori_loop(0, 257, zero_body, 0)

        def count_body(j, c, tab=tab):
            key = mkeys[pl.ds(j, 16)][0]
            t = lax.shift_right_logical(key, 21)
            ends[tab, t] = ends[tab, t] + 1
            return c
        lax.fori_loop(0, m, count_body, 0)

        def prefix_body(k, ssum, tab=tab):
            s2 = ssum + ends[tab, k]
            ends[tab, k] = s2
            return s2
        lax.fori_loop(0, nt, prefix_body, jnp.int32(0))

        def place_body(j, c, tab=tab):
            key = mkeys[pl.ds(j, 16)][0]
            t = lax.shift_right_logical(key, 21)
            pp = ends[tab, t] - 1
            ends[tab, t] = pp
            plsc.store_scatter(aidx,
                               [jnp.full((16,), tab * B16, jnp.int32) + pp],
                               jnp.full((16,), key, jnp.int32),
                               mask=(lane == 0))
            return c
        lax.fori_loop(0, m, place_body, 0)
        ends[tab, nt] = m

    for tab in range(2):
        tab_hbm = utab_hbm if tab == 0 else itab_hbm
        nc = lax.div(nt + 1, 2)

        def fire_chunk(c, r, tab_hbm=tab_hbm, slab_sems=slab_sems):
            off = pl.multiple_of((t0 + 2 * c) * 128, 128)
            for rr in range(4):
                @pl.when(r == rr)
                def _f(rr=rr):
                    pltpu.make_async_copy(
                        tab_hbm.at[:, pl.ds(off, 256)], slab.at[rr],
                        slab_sems[rr]).start()

        def wait_chunk(r, tab_hbm=tab_hbm, slab_sems=slab_sems):
            for rr in range(4):
                @pl.when(r == rr)
                def _w(rr=rr):
                    pltpu.make_async_copy(
                        tab_hbm.at[:, pl.ds(0, 256)], slab.at[rr],
                        slab_sems[rr]).wait()

        for k in range(3):
            @pl.when(k < nc)
            def _prime(k=k):
                fire_chunk(jnp.int32(k), jnp.int32(k))

        def chunk_body(c, used, tab=tab, tab_hbm=tab_hbm):
            r = lax.rem(c, 4)

            @pl.when(c + 3 < nc)
            def _fire_ahead():
                fire_chunk(c + 3, lax.rem(c + 3, 4))

            wait_chunk(r)

            lo = ends[tab, 2 * c]
            hi = ends[tab, jnp.minimum(2 * c + 2, nt)]
            bufv = jnp.full((16,), 0, jnp.int32) + r

            def group_body(g, used2, tab=tab, c=c):
                j0 = lo + g * RING
                new_used = used2
                for s in range(RING):
                    j = j0 + s

                    @pl.when(j < hi)
                    def _elem(s=s, j=j):
                        @pl.when((used2 >> s) & 1 == 1)
                        def _reuse_wait():
                            pltpu.make_async_copy(
                                cols.at[s], out_hbm.at[tab, 0],
                                ring_sems[s]).wait()

                        key = aidx[pl.ds(tab * B16 + j, 16)][0]
                        rel = lax.shift_right_logical(key, 21)
                        l2 = ((rel - 2 * c) * 128
                              + (lax.shift_right_logical(key, 14) & 127))
                        b = key & 16383
                        lv = jnp.full((16,), 0, jnp.int32) + l2
                        for q in range(4):
                            jv = q * 16 + lane
                            vals = plsc.load_gather(slab, [bufv, jv, lv])
                            cols[s, pl.ds(q * 16, 16)] = vals
                        pltpu.make_async_copy(
                            cols.at[s], out_hbm.at[tab, b],
                            ring_sems[s]).start()

                    new_used = jnp.where(j < hi, new_used | (1 << s),
                                         new_used)
                return new_used

            n_groups = lax.div(hi - lo + (RING - 1), RING)
            return lax.fori_loop(0, n_groups, group_body, used)

        used_f = lax.fori_loop(0, nc, chunk_body, jnp.int32(0))

        for s in range(RING):
            @pl.when((used_f >> s) & 1 == 1)
            def _drain(s=s):
                pltpu.make_async_copy(
                    cols.at[s], out_hbm.at[tab, 0], ring_sems[s]).wait()


@functools.partial(
    pl.kernel,
    out_type=jax.ShapeDtypeStruct((B,), jnp.float32),
    mesh=_mesh,
    scratch_types=[
        pltpu.VMEM((128, 128), jnp.float32),
        pltpu.VMEM((128, 128), jnp.float32),
        pltpu.VMEM((BPW,), jnp.float32),
    ],
    compiler_params=_params,
)
def _mf_dot(packed_hbm, out_hbm, uslab, islab, outv):
    wid = lax.axis_index("s") * NUM_CORES + lax.axis_index("c")
    base = wid * BPW
    lane = lax.iota(jnp.int32, 16)

    for h in range(BPW // 128):
        pltpu.sync_copy(packed_hbm.at[0, pl.ds(base + h * 128, 128), :], uslab)
        pltpu.sync_copy(packed_hbm.at[1, pl.ds(base + h * 128, 128), :], islab)

        def group_body(g, carry, h=h):
            r16 = g * LANES + lane
            acc = jnp.zeros((16,), jnp.float32)
            for j in range(F):
                cj = jnp.full((16,), j, jnp.int32)
                acc = acc + (plsc.load_gather(uslab, [r16, cj])
                             * plsc.load_gather(islab, [r16, cj]))
            outv[pl.ds(h * 128 + g * LANES, LANES)] = acc
            return carry

        lax.fori_loop(0, 128 // LANES, group_body, 0)

    pltpu.sync_copy(outv, out_hbm.at[pl.ds(base, BPW)])


def kernel(user, item, user_table, item_table):
    packed = _mf_extract(user.astype(jnp.int32), item.astype(jnp.int32),
                         user_table.T, item_table.T)
    return _mf_dot(packed)
